# baseline (speedup 1.0000x reference)
import functools

import jax
import jax.numpy as jnp
from jax import lax
from jax.experimental import pallas as pl
from jax.experimental.pallas import tpu as pltpu
from jax.experimental.pallas import tpu_sc as plsc

_HID = 128
_EPS = 0.1
_NC, _NS, _NW = 2, 16, 32
_CH = 128
_ECH = 64


def _chunk(n, cap, mult=8):
    best = mult
    c = mult
    while c <= cap:
        if n % c == 0:
            best = c
        c += mult
    return best


def _pad_rows(x, rows):
    return jnp.pad(x, ((0, rows - x.shape[0]),) + ((0, 0),) * (x.ndim - 1))


def _pad_idx(idx, n, fill):
    return jnp.pad(idx.astype(jnp.int32), (0, n - idx.shape[0]),
                   constant_values=fill)


def _mlp2(X, W1, b1, W2, b2, BM):
    M, K = X.shape
    H = W1.shape[1]
    N = W2.shape[1]

    def body(x_ref, w1_ref, b1_ref, w2_ref, b2_ref, o_ref):
        h = jnp.maximum(
            jnp.dot(x_ref[...], w1_ref[...],
                    preferred_element_type=jnp.float32) + b1_ref[...], 0.0)
        o_ref[...] = jnp.dot(h, w2_ref[...],
                             preferred_element_type=jnp.float32) + b2_ref[...]

    return pl.pallas_call(
        body,
        grid=(M // BM,),
        in_specs=[pl.BlockSpec((BM, K), lambda i: (i, 0)),
                  pl.BlockSpec((K, H), lambda i: (0, 0)),
                  pl.BlockSpec((1, H), lambda i: (0, 0)),
                  pl.BlockSpec((H, N), lambda i: (0, 0)),
                  pl.BlockSpec((1, N), lambda i: (0, 0))],
        out_specs=pl.BlockSpec((BM, N), lambda i: (i, 0)),
        out_shape=jax.ShapeDtypeStruct((M, N), jnp.float32),
    )(X, W1, b1.reshape(1, -1), W2, b2.reshape(1, -1))


def _edge_c(X, V1, c1, U01, v01, BM=1024):
    M, K = X.shape
    H = V1.shape[1]

    def body(x_ref, v1_ref, c1_ref, u_ref, v_ref, o0_ref, o1_ref):
        h = jnp.maximum(
            jnp.dot(x_ref[...], v1_ref[...],
                    preferred_element_type=jnp.float32) + c1_ref[...], 0.0)
        z = jnp.dot(h, u_ref[...], preferred_element_type=jnp.float32) + v_ref[...]
        o0_ref[...] = z[:, :128]
        o1_ref[...] = z[:, 128:]

    return pl.pallas_call(
        body,
        grid=(M // BM,),
        in_specs=[pl.BlockSpec((BM, K), lambda i: (i, 0)),
                  pl.BlockSpec((K, H), lambda i: (0, 0)),
                  pl.BlockSpec((1, H), lambda i: (0, 0)),
                  pl.BlockSpec((H, 256), lambda i: (0, 0)),
                  pl.BlockSpec((1, 256), lambda i: (0, 0))],
        out_specs=[pl.BlockSpec((BM, 128), lambda i: (i, 0)),
                   pl.BlockSpec((BM, 128), lambda i: (i, 0))],
        out_shape=[jax.ShapeDtypeStruct((M, 128), jnp.float32),
                   jax.ShapeDtypeStruct((M, 128), jnp.float32)],
    )(X, V1, c1.reshape(1, -1), U01, v01.reshape(1, -1))


def _matmul_multi(X, Wcat, bias, nout, BM):
    M, K = X.shape
    N = Wcat.shape[1]

    def body(x_ref, w_ref, b_ref, *o_refs):
        z = jnp.dot(x_ref[...], w_ref[...],
                    preferred_element_type=jnp.float32) + b_ref[...]
        for k in range(nout):
            o_refs[k][...] = z[:, 128 * k:128 * (k + 1)]

    return pl.pallas_call(
        body,
        grid=(M // BM,),
        in_specs=[pl.BlockSpec((BM, K), lambda i: (i, 0)),
                  pl.BlockSpec((K, N), lambda i: (0, 0)),
                  pl.BlockSpec((1, N), lambda i: (0, 0))],
        out_specs=[pl.BlockSpec((BM, 128), lambda i: (i, 0))] * nout,
        out_shape=[jax.ShapeDtypeStruct((M, 128), jnp.float32)] * nout,
    )(X, Wcat, bias.reshape(1, -1))


def _update(x, Ss, W2s, degs, b2s, cmul, relu_out, BM):
    M = x.shape[0]
    nk = len(Ss)

    def body(x_ref, *refs):
        s_refs = refs[:nk]
        w_refs = refs[nk:2 * nk]
        d_refs = refs[2 * nk:3 * nk]
        b_refs = refs[3 * nk:4 * nk]
        o_ref = refs[4 * nk]
        out = cmul * x_ref[...]
        for k in range(nk):
            out = out + jnp.dot(s_refs[k][...], w_refs[k][...],
                                preferred_element_type=jnp.float32)
            out = out + d_refs[k][...] * b_refs[k][...]
        if relu_out:
            o_ref[...] = jnp.maximum(x_ref[...] + out, 0.0)
        else:
            o_ref[...] = out

    in_specs = [pl.BlockSpec((BM, 128), lambda i: (i, 0))]
    in_specs += [pl.BlockSpec((BM, 128), lambda i: (i, 0))] * nk
    in_specs += [pl.BlockSpec((128, 128), lambda i: (0, 0))] * nk
    in_specs += [pl.BlockSpec((BM, 1), lambda i: (i, 0))] * nk
    in_specs += [pl.BlockSpec((1, 128), lambda i: (0, 0))] * nk
    return pl.pallas_call(
        body,
        grid=(M // BM,),
        in_specs=in_specs,
        out_specs=pl.BlockSpec((BM, 128), lambda i: (i, 0)),
        out_shape=jax.ShapeDtypeStruct((M, 128), jnp.float32),
    )(x, *Ss, *W2s, *degs, *b2s)


_MESH = dict(core_axis_name="c", subcore_axis_name="s",
             num_cores=_NC, num_subcores=_NS)


def _gather_rows(table, idx):
    B = idx.shape[0]
    D = table.shape[1]
    bpw = B // _NW
    gch = _chunk(bpw, _CH)
    nch = bpw // gch
    mesh = plsc.VectorSubcoreMesh(**_MESH)

    @functools.partial(
        pl.kernel, mesh=mesh,
        out_type=jax.ShapeDtypeStruct((B, D), jnp.float32),
        scratch_types=[pltpu.VMEM((gch,), jnp.int32),
                       pltpu.VMEM((gch, D), jnp.float32),
                       pltpu.SemaphoreType.DMA])
    def k(tab_hbm, idx_hbm, out_hbm, idxv, rows, sem):
        wid = lax.axis_index("c") * _NS + lax.axis_index("s")
        base0 = wid * bpw

        def ch(i, _):
            base = base0 + i * gch
            pltpu.sync_copy(idx_hbm.at[pl.ds(base, gch)], idxv)
            pltpu.async_copy(tab_hbm.at[idxv], rows, sem).wait()
            pltpu.sync_copy(rows, out_hbm.at[pl.ds(base, gch)])
            return 0

        lax.fori_loop(0, nch, ch, 0)

    return k(table, idx)


def _edge_pair(jobs0, jobs1, with_deg):
    n_acc = max(jb[4] for jb in jobs0 + jobs1)
    rpt = n_acc // _NS
    zr = _chunk(rpt, 16, 1)
    mesh = plsc.VectorSubcoreMesh(**_MESH)

    out_type = [jax.ShapeDtypeStruct((jb[4], 128), jnp.float32)
                for jb in jobs0 + jobs1]
    scratch = [pltpu.VMEM((2, 2, _ECH), jnp.int32),
               pltpu.VMEM((2, _ECH, 128), jnp.float32),
               pltpu.VMEM((2, _ECH, 128), jnp.float32),
               pltpu.VMEM((zr, 128), jnp.float32),
               pltpu.VMEM_SHARED((n_acc, 128), jnp.float32)]
    scratch += [pltpu.SemaphoreType.DMA] * 12
    if with_deg:
        out_type += [jax.ShapeDtypeStruct((jb[4],), jnp.float32)
                     for jb in jobs0 + jobs1]
        scratch.insert(4, pltpu.VMEM((_ECH,), jnp.float32))
        scratch.insert(5, pltpu.VMEM_SHARED((n_acc,), jnp.float32))
        scratch.insert(6, pltpu.VMEM((n_acc // _NS,), jnp.float32))

    nj = len(jobs0) + len(jobs1)

    @functools.partial(pl.kernel, mesh=mesh, out_type=out_type,
                       scratch_types=scratch)
    def k(*args):
        ins = args[:4 * nj]
        o_hbms = list(args[4 * nj:5 * nj])
        rest = args[5 * nj:]
        if with_deg:
            d_hbms = list(rest[:nj])
            (dsi, buft, bufs, zbuf, ones, dacc, dvbuf,
             acc, *sems) = rest[nj:]
        else:
            d_hbms = [None] * nj
            dacc = dvbuf = ones = None
            dsi, buft, bufs, zbuf, acc, *sems = rest
        cid = lax.axis_index("c")
        sid = lax.axis_index("s")
        zero16 = jnp.zeros((16,), jnp.float32)

        def zrow(i, _):
            for j in range(8):
                zbuf[i, pl.ds(j * 16, 16)] = zero16
            return 0

        lax.fori_loop(0, zr, zrow, 0)
        if with_deg:
            one16 = jnp.full((16,), 1.0, jnp.float32)

            def orow(i, _):
                ones[pl.ds(i * 16, 16)] = one16
                return 0

            lax.fori_loop(0, _ECH // 16, orow, 0)

        def zero_acc():
            def zcp(i, _):
                pltpu.sync_copy(zbuf, acc.at[pl.ds(sid * rpt + i * zr, zr)])
                return 0

            lax.fori_loop(0, rpt // zr, zcp, 0)
            if with_deg:
                dz = _chunk(rpt, 128, 8)

                def dzcp(i, _):
                    pltpu.sync_copy(zbuf.at[0, pl.ds(0, dz)],
                                    dacc.at[pl.ds(sid * rpt + i * dz, dz)])
                    return 0

                lax.fori_loop(0, rpt // dz, dzcp, 0)
            plsc.subcore_barrier()

        def run(a_hbm, b_hbm, c_hbm, ds_hbm, o_hbm, d_hbm, E, n_real):
            per_tile = E // _NS
            n_chunks = per_tile // _ECH
            base0 = sid * per_tile

            def fetch(s, base):
                pltpu.async_copy(ds_hbm.at[base // _ECH], dsi.at[s],
                                 sems[6 * s + 0])
                pltpu.async_copy(c_hbm.at[pl.ds(base, _ECH)], buft.at[s],
                                 sems[6 * s + 2])

            def wait_scatter(s):
                pltpu.make_async_copy(bufs.at[s], acc.at[dsi.at[s, 0]],
                                      sems[6 * s + 5]).wait()

            def wait_deg(s):
                pltpu.make_async_copy(ones, dacc.at[dsi.at[s, 0]],
                                      sems[6 * s + 1]).wait()

            def process_a(s, prev_done):
                @pl.when(prev_done)
                def _():
                    wait_scatter(s)
                    if with_deg:
                        wait_deg(s)
                pltpu.make_async_copy(ds_hbm.at[0], dsi.at[s],
                                      sems[6 * s + 0]).wait()
                pltpu.make_async_copy(c_hbm.at[pl.ds(0, _ECH)], buft.at[s],
                                      sems[6 * s + 2]).wait()
                if with_deg:
                    pltpu.async_copy(ones, dacc.at[dsi.at[s, 0]],
                                     sems[6 * s + 1], add=True)
                pltpu.async_copy(a_hbm.at[dsi.at[s, 0]], buft.at[s],
                                 sems[6 * s + 3], add=True)
                pltpu.async_copy(b_hbm.at[dsi.at[s, 1]], buft.at[s],
                                 sems[6 * s + 4], add=True)

            def process_b(s):
                pltpu.make_async_copy(a_hbm.at[dsi.at[s, 0]], buft.at[s],
                                      sems[6 * s + 3]).wait()
                pltpu.make_async_copy(b_hbm.at[dsi.at[s, 1]], buft.at[s],
                                      sems[6 * s + 4]).wait()

                def row(r, _):
                    for u in range(4):
                        for j in range(8):
                            sl = pl.ds(j * 16, 16)
                            bufs[s, 4 * r + u, sl] = jnp.maximum(
                                buft[s, 4 * r + u, sl], 0.0)
                    return 0

                lax.fori_loop(0, _ECH // 4, row, 0)
                pltpu.async_copy(bufs.at[s], acc.at[dsi.at[s, 0]],
                                 sems[6 * s + 5], add=True)

            fetch(0, base0)
            fetch(1, base0 + _ECH)
            process_a(0, False)

            def body(i, _):
                b2 = base0 + 2 * i * _ECH
                process_a(1, i > 0)
                process_b(0)

                @pl.when(2 * i + 2 < n_chunks)
                def _():
                    fetch(0, b2 + 2 * _ECH)
                    process_a(0, True)

                process_b(1)

                @pl.when(2 * i + 3 < n_chunks)
                def _():
                    fetch(1, b2 + 3 * _ECH)

                return 0

            lax.fori_loop(0, n_chunks // 2, body, 0)
            wait_scatter(0)
            wait_scatter(1)
            if with_deg:
                wait_deg(0)
                wait_deg(1)
            plsc.subcore_barrier()
            dpt = n_real // _NS
            dr = _chunk(dpt, 128, 1)

            def drain(i, _):
                r0 = sid * dpt + i * dr
                pltpu.sync_copy(acc.at[pl.ds(r0, dr)],
                                o_hbm.at[pl.ds(r0, dr)])
                return 0

            lax.fori_loop(0, dpt // dr, drain, 0)
            if with_deg:
                r0 = sid * dpt
                pltpu.sync_copy(dacc.at[pl.ds(r0, dpt)],
                                dvbuf.at[pl.ds(0, dpt)])
                pltpu.sync_copy(dvbuf.at[pl.ds(0, dpt)],
                                d_hbm.at[pl.ds(r0, dpt)])

        for ji, core in ([(a, 0) for a in range(len(jobs0))] +
                         [(len(jobs0) + a, 1) for a in range(len(jobs1))]):
            jb = (jobs0 + jobs1)[ji]
            E = jb[3].shape[0] * _ECH

            @pl.when(cid == core)
            def _(ji=ji, E=E, n_real=jb[4]):
                zero_acc()
                run(ins[4 * ji], ins[4 * ji + 1], ins[4 * ji + 2],
                    ins[4 * ji + 3], o_hbms[ji], d_hbms[ji], E, n_real)

    flat = [x for jb in jobs0 + jobs1 for x in jb[:4]]
    return k(*flat)


def _pred_gather(Pu, Pb, iu, ib, w, bvec):
    L = iu.shape[0]
    per = L // _NW
    nch = per // _CH
    mesh = plsc.VectorSubcoreMesh(**_MESH)

    @functools.partial(
        pl.kernel, mesh=mesh,
        out_type=jax.ShapeDtypeStruct((L, 16), jnp.float32),
        scratch_types=[pltpu.VMEM((128,), jnp.float32),
                       pltpu.VMEM((16,), jnp.float32),
                       pltpu.VMEM((_CH,), jnp.int32),
                       pltpu.VMEM((_CH,), jnp.int32),
                       pltpu.VMEM((_CH, 128), jnp.float32),
                       pltpu.VMEM((_CH, 128), jnp.float32),
                       pltpu.VMEM((_CH, 16), jnp.float32),
                       pltpu.SemaphoreType.DMA,
                       pltpu.SemaphoreType.DMA])
    def k(pu_hbm, pb_hbm, iu_hbm, ib_hbm, w_hbm, b_hbm, out_hbm,
          wbuf, bbuf, iuv, ibv, bu, bb, pbuf, s1, s2):
        wid = lax.axis_index("c") * _NS + lax.axis_index("s")
        pltpu.sync_copy(w_hbm, wbuf)
        pltpu.sync_copy(b_hbm, bbuf)
        lane = lax.iota(jnp.int32, 16)
        bv = bbuf[pl.ds(0, 16)]
        acc0 = jnp.where(lane == 0, bv, 0.0)
        wv = [wbuf[pl.ds(j * 16, 16)] for j in range(8)]
        base0 = wid * per

        def ch(i, _):
            base = base0 + i * _CH
            pltpu.sync_copy(iu_hbm.at[pl.ds(base, _CH)], iuv)
            pltpu.sync_copy(ib_hbm.at[pl.ds(base, _CH)], ibv)
            cu = pltpu.async_copy(pu_hbm.at[iuv], bu, s1)
            cb = pltpu.async_copy(pb_hbm.at[ibv], bb, s2)
            cu.wait()
            cb.wait()

            def rowf(ri, _):
                acc = acc0
                for j in range(8):
                    sl = pl.ds(j * 16, 16)
                    q = jnp.maximum(bu[ri, sl] + bb[ri, sl], 0.0)
                    acc = acc + q * wv[j]
                pbuf[ri] = acc
                return 0

            lax.fori_loop(0, _CH, rowf, 0)
            pltpu.sync_copy(pbuf, out_hbm.at[pl.ds(base, _CH)])
            return 0

        lax.fori_loop(0, nch, ch, 0)

    return k(Pu, Pb, iu, ib, w, bvec)


def _pred_reduce(P, BM=1024):
    L = P.shape[0]

    def body(p_ref, o_ref):
        o_ref[...] = jnp.sum(p_ref[...], axis=1, keepdims=True)

    return pl.pallas_call(
        body,
        grid=(L // BM,),
        in_specs=[pl.BlockSpec((BM, 16), lambda i: (i, 0))],
        out_specs=pl.BlockSpec((BM, 1), lambda i: (i, 0)),
        out_shape=jax.ShapeDtypeStruct((L, 1), jnp.float32),
    )(P)


_ETS = ['ub', 'bu', 'bg', 'gb']
_TOPO = {'ub': ('user', 'book'), 'bu': ('book', 'user'),
         'bg': ('book', 'genre'), 'gb': ('genre', 'book')}


def kernel(params, n_id_user, n_id_book, n_id_genre, src_ub, dst_ub, src_bu,
           dst_bu, src_bg, dst_bg, src_gb, dst_gb, eli_user, eli_book,
           ea_ub, ea_bu, ea_bg, ea_gb):
    p = params
    zeros128 = jnp.zeros((128,), jnp.float32)

    ncnt = {'user': n_id_user.shape[0], 'book': n_id_book.shape[0],
            'genre': n_id_genre.shape[0]}
    npad = {t: ((ncnt[t] + 1 + 255) // 256) * 256 for t in ncnt}

    src_e = {'ub': src_ub, 'bu': src_bu, 'bg': src_bg, 'gb': src_gb}
    dst_e = {'ub': dst_ub, 'bu': dst_bu, 'bg': dst_bg, 'gb': dst_gb}
    ea_e = {'ub': ea_ub, 'bu': ea_bu, 'bg': ea_bg, 'gb': ea_gb}
    ds_pack = {}
    epad = {}
    for e in _ETS:
        E = src_e[e].shape[0]
        Ep = ((E + _NS * _ECH * 2 - 1) // (_NS * _ECH * 2)) * (_NS * _ECH * 2)
        epad[e] = Ep
        d = _TOPO[e][1]
        src_p = _pad_idx(src_e[e], Ep, 0)
        dst_p = _pad_idx(dst_e[e], Ep, ncnt[d])
        ds_pack[e] = jnp.stack([dst_p.reshape(-1, _ECH),
                                src_p.reshape(-1, _ECH)], axis=1)
        ea_e[e] = _pad_rows(ea_e[e], Ep)

    Mu = _mlp2(p['emb_user'], p['init_node']['user']['W1'],
               p['init_node']['user']['b1'], p['init_node']['user']['W2'],
               p['init_node']['user']['b2'], 1000)
    Mb = _mlp2(p['emb_book'], p['init_node']['book']['W1'],
               p['init_node']['book']['b1'], p['init_node']['book']['W2'],
               p['init_node']['book']['b2'], 1000)
    Mg = _mlp2(p['emb_book'], p['init_node']['genre']['W1'],
               p['init_node']['genre']['b1'], p['init_node']['genre']['W2'],
               p['init_node']['genre']['b2'], 1000)
    x = {'user': _gather_rows(Mu, _pad_idx(n_id_user, npad['user'], 0)),
         'book': _gather_rows(Mb, _pad_idx(n_id_book, npad['book'], 0)),
         'genre': _gather_rows(Mg, _pad_idx(n_id_genre, npad['genre'], 0))}

    C = {}
    for e in _ETS:
        ep = p['init_edge'][e]
        Us, vs = [], []
        for li in range(2):
            W1c = p['convs'][li][e]['W1'][256:384]
            Us.append(ep['W2'] @ W1c)
            vs.append(ep['b2'] @ W1c + p['convs'][li][e]['b1'])
        C[e] = _edge_c(ea_e[e], ep['W1'], ep['b1'],
                       jnp.concatenate(Us, axis=1), jnp.concatenate(vs))

    degs = {}
    for li in range(2):
        cw = {e: p['convs'][li][e]['W1'] for e in _ETS}
        b_ub, a_bu = _matmul_multi(
            x['user'], jnp.concatenate([cw['ub'][128:256], cw['bu'][:128]],
                                       axis=1), zeros128.repeat(2), 2, 1024)
        a_ub, b_bu, b_bg, a_gb = _matmul_multi(
            x['book'], jnp.concatenate([cw['ub'][:128], cw['bu'][128:256],
                                        cw['bg'][128:256], cw['gb'][:128]],
                                       axis=1), zeros128.repeat(4), 4, 1024)
        a_bg, b_gb = _matmul_multi(
            x['genre'], jnp.concatenate([cw['bg'][:128], cw['gb'][128:256]],
                                        axis=1), zeros128.repeat(2), 2,
            _chunk(npad['genre'], 1024))
        S = {}
        res = _edge_pair(
            [(a_ub, b_ub, C['ub'][li], ds_pack['ub'], npad['book']),
             (a_gb, b_gb, C['gb'][li], ds_pack['gb'], npad['book'])],
            [(a_bu, b_bu, C['bu'][li], ds_pack['bu'], npad['user']),
             (a_bg, b_bg, C['bg'][li], ds_pack['bg'], npad['genre'])],
            li == 0)
        S['ub'], S['gb'], S['bu'], S['bg'] = res[:4]
        if li == 0:
            for key, dg in zip(['ub', 'gb', 'bu', 'bg'], res[4:]):
                degs[key] = dg.reshape(-1, 1)
        w2 = {e: p['convs'][li][e]['W2'] for e in _ETS}
        b2 = {e: p['convs'][li][e]['b2'].reshape(1, -1) for e in _ETS}
        relu_out = li < 1
        x = {'book': _update(x['book'], [S['ub'], S['gb']],
                             [w2['ub'], w2['gb']], [degs['ub'], degs['gb']],
                             [b2['ub'], b2['gb']],
                             2.0 * (1.0 + _EPS), relu_out, 1024),
             'user': _update(x['user'], [S['bu']], [w2['bu']], [degs['bu']],
                             [b2['bu']], 1.0 + _EPS, relu_out, 1024),
             'genre': _update(x['genre'], [S['bg']], [w2['bg']], [degs['bg']],
                              [b2['bg']], 1.0 + _EPS, relu_out,
                              _chunk(npad['genre'], 1024))}

    L = eli_user.shape[0]
    Lp = ((L + _NW * _CH - 1) // (_NW * _CH)) * (_NW * _CH)
    Pu, = _matmul_multi(x['user'], p['cls']['W1'][:128], p['cls']['b1'],
                        1, 1024)
    Pb, = _matmul_multi(x['book'], p['cls']['W1'][128:], zeros128, 1, 1024)
    part = _pred_gather(Pu, Pb, _pad_idx(eli_user, Lp, 0),
                        _pad_idx(eli_book, Lp, 0), p['cls']['W2'][:, 0],
                        jnp.full((16,), p['cls']['b2'][0], jnp.float32))
    pred = _pred_reduce(part).reshape(-1)

    return (pred[:L], x['user'][:ncnt['user']], x['book'][:ncnt['book']],
            x['genre'][:ncnt['genre']])


# speedup vs baseline: 1.1090x; 1.1090x over previous
import functools

import jax
import jax.numpy as jnp
from jax import lax
from jax.experimental import pallas as pl
from jax.experimental.pallas import tpu as pltpu
from jax.experimental.pallas import tpu_sc as plsc

_HID = 128
_EPS = 0.1
_NC, _NS, _NW = 2, 16, 32
_CH = 128
_ECH = 64


def _chunk(n, cap, mult=8):
    best = mult
    c = mult
    while c <= cap:
        if n % c == 0:
            best = c
        c += mult
    return best


def _pad_rows(x, rows):
    return jnp.pad(x, ((0, rows - x.shape[0]),) + ((0, 0),) * (x.ndim - 1))


def _pad_idx(idx, n, fill):
    return jnp.pad(idx.astype(jnp.int32), (0, n - idx.shape[0]),
                   constant_values=fill)


def _mlp2(X, W1, b1, W2, b2, BM):
    M, K = X.shape
    H = W1.shape[1]
    N = W2.shape[1]

    def body(x_ref, w1_ref, b1_ref, w2_ref, b2_ref, o_ref):
        h = jnp.maximum(
            jnp.dot(x_ref[...], w1_ref[...],
                    preferred_element_type=jnp.float32) + b1_ref[...], 0.0)
        o_ref[...] = jnp.dot(h, w2_ref[...],
                             preferred_element_type=jnp.float32) + b2_ref[...]

    return pl.pallas_call(
        body,
        grid=(M // BM,),
        in_specs=[pl.BlockSpec((BM, K), lambda i: (i, 0)),
                  pl.BlockSpec((K, H), lambda i: (0, 0)),
                  pl.BlockSpec((1, H), lambda i: (0, 0)),
                  pl.BlockSpec((H, N), lambda i: (0, 0)),
                  pl.BlockSpec((1, N), lambda i: (0, 0))],
        out_specs=pl.BlockSpec((BM, N), lambda i: (i, 0)),
        out_shape=jax.ShapeDtypeStruct((M, N), jnp.float32),
    )(X, W1, b1.reshape(1, -1), W2, b2.reshape(1, -1))


def _edge_c(X, V1, c1, U01, v01, BM=1024):
    M, K = X.shape
    H = V1.shape[1]

    def body(x_ref, v1_ref, c1_ref, u_ref, v_ref, o0_ref, o1_ref):
        h = jnp.maximum(
            jnp.dot(x_ref[...], v1_ref[...],
                    preferred_element_type=jnp.float32) + c1_ref[...], 0.0)
        z = jnp.dot(h, u_ref[...], preferred_element_type=jnp.float32) + v_ref[...]
        o0_ref[...] = z[:, :128]
        o1_ref[...] = z[:, 128:]

    return pl.pallas_call(
        body,
        grid=(M // BM,),
        in_specs=[pl.BlockSpec((BM, K), lambda i: (i, 0)),
                  pl.BlockSpec((K, H), lambda i: (0, 0)),
                  pl.BlockSpec((1, H), lambda i: (0, 0)),
                  pl.BlockSpec((H, 256), lambda i: (0, 0)),
                  pl.BlockSpec((1, 256), lambda i: (0, 0))],
        out_specs=[pl.BlockSpec((BM, 128), lambda i: (i, 0)),
                   pl.BlockSpec((BM, 128), lambda i: (i, 0))],
        out_shape=[jax.ShapeDtypeStruct((M, 128), jnp.float32),
                   jax.ShapeDtypeStruct((M, 128), jnp.float32)],
    )(X, V1, c1.reshape(1, -1), U01, v01.reshape(1, -1))


def _matmul_multi(X, Wcat, bias, nout, BM):
    M, K = X.shape
    N = Wcat.shape[1]

    def body(x_ref, w_ref, b_ref, *o_refs):
        z = jnp.dot(x_ref[...], w_ref[...],
                    preferred_element_type=jnp.float32) + b_ref[...]
        for k in range(nout):
            o_refs[k][...] = z[:, 128 * k:128 * (k + 1)]

    return pl.pallas_call(
        body,
        grid=(M // BM,),
        in_specs=[pl.BlockSpec((BM, K), lambda i: (i, 0)),
                  pl.BlockSpec((K, N), lambda i: (0, 0)),
                  pl.BlockSpec((1, N), lambda i: (0, 0))],
        out_specs=[pl.BlockSpec((BM, 128), lambda i: (i, 0))] * nout,
        out_shape=[jax.ShapeDtypeStruct((M, 128), jnp.float32)] * nout,
    )(X, Wcat, bias.reshape(1, -1))


def _update(x, Ss, W2s, degs, b2s, cmul, relu_out, BM):
    M = x.shape[0]
    nk = len(Ss)

    def body(x_ref, *refs):
        s_refs = refs[:nk]
        w_refs = refs[nk:2 * nk]
        d_refs = refs[2 * nk:3 * nk]
        b_refs = refs[3 * nk:4 * nk]
        o_ref = refs[4 * nk]
        out = cmul * x_ref[...]
        for k in range(nk):
            out = out + jnp.dot(s_refs[k][...], w_refs[k][...],
                                preferred_element_type=jnp.float32)
            out = out + d_refs[k][...] * b_refs[k][...]
        if relu_out:
            o_ref[...] = jnp.maximum(x_ref[...] + out, 0.0)
        else:
            o_ref[...] = out

    in_specs = [pl.BlockSpec((BM, 128), lambda i: (i, 0))]
    in_specs += [pl.BlockSpec((BM, 128), lambda i: (i, 0))] * nk
    in_specs += [pl.BlockSpec((128, 128), lambda i: (0, 0))] * nk
    in_specs += [pl.BlockSpec((BM, 1), lambda i: (i, 0))] * nk
    in_specs += [pl.BlockSpec((1, 128), lambda i: (0, 0))] * nk
    return pl.pallas_call(
        body,
        grid=(M // BM,),
        in_specs=in_specs,
        out_specs=pl.BlockSpec((BM, 128), lambda i: (i, 0)),
        out_shape=jax.ShapeDtypeStruct((M, 128), jnp.float32),
    )(x, *Ss, *W2s, *degs, *b2s)


_MESH = dict(core_axis_name="c", subcore_axis_name="s",
             num_cores=_NC, num_subcores=_NS)


def _gather_rows(table, idx):
    B = idx.shape[0]
    D = table.shape[1]
    bpw = B // _NW
    gch = _chunk(bpw, _CH)
    nch = bpw // gch
    mesh = plsc.VectorSubcoreMesh(**_MESH)

    @functools.partial(
        pl.kernel, mesh=mesh,
        out_type=jax.ShapeDtypeStruct((B, D), jnp.float32),
        scratch_types=[pltpu.VMEM((gch,), jnp.int32),
                       pltpu.VMEM((gch, D), jnp.float32),
                       pltpu.SemaphoreType.DMA])
    def k(tab_hbm, idx_hbm, out_hbm, idxv, rows, sem):
        wid = lax.axis_index("c") * _NS + lax.axis_index("s")
        base0 = wid * bpw

        def ch(i, _):
            base = base0 + i * gch
            pltpu.sync_copy(idx_hbm.at[pl.ds(base, gch)], idxv)
            pltpu.async_copy(tab_hbm.at[idxv], rows, sem).wait()
            pltpu.sync_copy(rows, out_hbm.at[pl.ds(base, gch)])
            return 0

        lax.fori_loop(0, nch, ch, 0)

    return k(table, idx)


def _edge_pair(args0, args1, with_deg):
    (A0, B0, C0, ds0, n0) = args0
    (A1, B1, C1, ds1, n1) = args1
    E0 = ds0.shape[0] * _ECH
    E1 = ds1.shape[0] * _ECH
    n_acc = max(n0, n1)
    rpt = n_acc // _NS
    zr = _chunk(rpt, 16, 1)
    mesh = plsc.VectorSubcoreMesh(**_MESH)

    out_type = [jax.ShapeDtypeStruct((n0, 128), jnp.float32),
                jax.ShapeDtypeStruct((n1, 128), jnp.float32)]
    scratch = [pltpu.VMEM((2, 2, _ECH), jnp.int32),
               pltpu.VMEM((2, _ECH, 128), jnp.float32),
               pltpu.VMEM((2, _ECH, 128), jnp.float32),
               pltpu.VMEM((zr, 128), jnp.float32),
               pltpu.VMEM_SHARED((n_acc, 128), jnp.float32)]
    scratch += [pltpu.SemaphoreType.DMA] * 12
    if with_deg:
        out_type += [jax.ShapeDtypeStruct((n0,), jnp.float32),
                     jax.ShapeDtypeStruct((n1,), jnp.float32)]
        scratch.insert(4, pltpu.VMEM((_ECH,), jnp.float32))
        scratch.insert(5, pltpu.VMEM_SHARED((n_acc,), jnp.float32))
        scratch.insert(6, pltpu.VMEM((n_acc // _NS,), jnp.float32))

    @functools.partial(pl.kernel, mesh=mesh, out_type=out_type,
                       scratch_types=scratch)
    def k(a0_hbm, b0_hbm, c0_hbm, ds0_hbm, a1_hbm, b1_hbm, c1_hbm, ds1_hbm,
          o0_hbm, o1_hbm, *rest):
        if with_deg:
            (d0_hbm, d1_hbm, dsi, buft, bufs, zbuf, ones, dacc, dvbuf,
             acc, *sems) = rest
        else:
            d0_hbm = d1_hbm = dacc = dvbuf = ones = None
            dsi, buft, bufs, zbuf, acc, *sems = rest
        cid = lax.axis_index("c")
        sid = lax.axis_index("s")
        zero16 = jnp.zeros((16,), jnp.float32)

        def zrow(i, _):
            for j in range(8):
                zbuf[i, pl.ds(j * 16, 16)] = zero16
            return 0

        lax.fori_loop(0, zr, zrow, 0)

        def zcp(i, _):
            pltpu.sync_copy(zbuf, acc.at[pl.ds(sid * rpt + i * zr, zr)])
            return 0

        lax.fori_loop(0, rpt // zr, zcp, 0)
        if with_deg:
            one16 = jnp.full((16,), 1.0, jnp.float32)

            def orow(i, _):
                ones[pl.ds(i * 16, 16)] = one16
                return 0

            lax.fori_loop(0, _ECH // 16, orow, 0)
            dz = _chunk(rpt, 128, 8)

            def dzcp(i, _):
                pltpu.sync_copy(zbuf.at[0, pl.ds(0, dz)],
                                dacc.at[pl.ds(sid * rpt + i * dz, dz)])
                return 0

            lax.fori_loop(0, rpt // dz, dzcp, 0)
        plsc.subcore_barrier()

        def run(a_hbm, b_hbm, c_hbm, ds_hbm, o_hbm, d_hbm, E, n_real):
            per_tile = E // _NS
            n_chunks = per_tile // _ECH
            base0 = sid * per_tile

            def fetch(s, base):
                pltpu.async_copy(ds_hbm.at[base // _ECH], dsi.at[s],
                                 sems[6 * s + 0])
                pltpu.async_copy(c_hbm.at[pl.ds(base, _ECH)], buft.at[s],
                                 sems[6 * s + 2])

            def wait_scatter(s):
                pltpu.make_async_copy(bufs.at[s], acc.at[dsi.at[s, 0]],
                                      sems[6 * s + 5]).wait()

            def wait_deg(s):
                pltpu.make_async_copy(ones, dacc.at[dsi.at[s, 0]],
                                      sems[6 * s + 1]).wait()

            def process_a(s, prev_done):
                @pl.when(prev_done)
                def _():
                    wait_scatter(s)
                    if with_deg:
                        wait_deg(s)
                pltpu.make_async_copy(ds_hbm.at[0], dsi.at[s],
                                      sems[6 * s + 0]).wait()
                pltpu.make_async_copy(c_hbm.at[pl.ds(0, _ECH)], buft.at[s],
                                      sems[6 * s + 2]).wait()
                if with_deg:
                    pltpu.async_copy(ones, dacc.at[dsi.at[s, 0]],
                                     sems[6 * s + 1], add=True)
                pltpu.async_copy(a_hbm.at[dsi.at[s, 0]], buft.at[s],
                                 sems[6 * s + 3], add=True)
                pltpu.async_copy(b_hbm.at[dsi.at[s, 1]], buft.at[s],
                                 sems[6 * s + 4], add=True)

            def process_b(s):
                pltpu.make_async_copy(a_hbm.at[dsi.at[s, 0]], buft.at[s],
                                      sems[6 * s + 3]).wait()
                pltpu.make_async_copy(b_hbm.at[dsi.at[s, 1]], buft.at[s],
                                      sems[6 * s + 4]).wait()

                def row(r, _):
                    for u in range(4):
                        for j in range(8):
                            sl = pl.ds(j * 16, 16)
                            bufs[s, 4 * r + u, sl] = jnp.maximum(
                                buft[s, 4 * r + u, sl], 0.0)
                    return 0

                lax.fori_loop(0, _ECH // 4, row, 0)
                pltpu.async_copy(bufs.at[s], acc.at[dsi.at[s, 0]],
                                 sems[6 * s + 5], add=True)

            fetch(0, base0)
            fetch(1, base0 + _ECH)
            process_a(0, False)

            def body(i, _):
                b2 = base0 + 2 * i * _ECH
                process_a(1, i > 0)
                process_b(0)

                @pl.when(2 * i + 2 < n_chunks)
                def _():
                    fetch(0, b2 + 2 * _ECH)
                    process_a(0, True)

                process_b(1)

                @pl.when(2 * i + 3 < n_chunks)
                def _():
                    fetch(1, b2 + 3 * _ECH)

                return 0

            lax.fori_loop(0, n_chunks // 2, body, 0)
            wait_scatter(0)
            wait_scatter(1)
            if with_deg:
                wait_deg(0)
                wait_deg(1)
            plsc.subcore_barrier()
            dpt = n_real // _NS
            dr = _chunk(dpt, 128, 1)

            def drain(i, _):
                r0 = sid * dpt + i * dr
                pltpu.sync_copy(acc.at[pl.ds(r0, dr)],
                                o_hbm.at[pl.ds(r0, dr)])
                return 0

            lax.fori_loop(0, dpt // dr, drain, 0)
            if with_deg:
                r0 = sid * dpt
                pltpu.sync_copy(dacc.at[pl.ds(r0, dpt)],
                                dvbuf.at[pl.ds(0, dpt)])
                pltpu.sync_copy(dvbuf.at[pl.ds(0, dpt)],
                                d_hbm.at[pl.ds(r0, dpt)])

        @pl.when(cid == 0)
        def _():
            run(a0_hbm, b0_hbm, c0_hbm, ds0_hbm, o0_hbm, d0_hbm, E0, n0)

        @pl.when(cid == 1)
        def _():
            run(a1_hbm, b1_hbm, c1_hbm, ds1_hbm, o1_hbm, d1_hbm, E1, n1)

    return k(A0, B0, C0, ds0, A1, B1, C1, ds1)


def _pred_gather(Pu, Pb, iu, ib, w, bvec):
    L = iu.shape[0]
    per = L // _NW
    nch = per // _CH
    mesh = plsc.VectorSubcoreMesh(**_MESH)

    @functools.partial(
        pl.kernel, mesh=mesh,
        out_type=jax.ShapeDtypeStruct((L, 16), jnp.float32),
        scratch_types=[pltpu.VMEM((128,), jnp.float32),
                       pltpu.VMEM((16,), jnp.float32),
                       pltpu.VMEM((_CH,), jnp.int32),
                       pltpu.VMEM((_CH,), jnp.int32),
                       pltpu.VMEM((_CH, 128), jnp.float32),
                       pltpu.VMEM((_CH, 128), jnp.float32),
                       pltpu.VMEM((_CH, 16), jnp.float32),
                       pltpu.SemaphoreType.DMA,
                       pltpu.SemaphoreType.DMA])
    def k(pu_hbm, pb_hbm, iu_hbm, ib_hbm, w_hbm, b_hbm, out_hbm,
          wbuf, bbuf, iuv, ibv, bu, bb, pbuf, s1, s2):
        wid = lax.axis_index("c") * _NS + lax.axis_index("s")
        pltpu.sync_copy(w_hbm, wbuf)
        pltpu.sync_copy(b_hbm, bbuf)
        lane = lax.iota(jnp.int32, 16)
        bv = bbuf[pl.ds(0, 16)]
        acc0 = jnp.where(lane == 0, bv, 0.0)
        wv = [wbuf[pl.ds(j * 16, 16)] for j in range(8)]
        base0 = wid * per

        def ch(i, _):
            base = base0 + i * _CH
            pltpu.sync_copy(iu_hbm.at[pl.ds(base, _CH)], iuv)
            pltpu.sync_copy(ib_hbm.at[pl.ds(base, _CH)], ibv)
            cu = pltpu.async_copy(pu_hbm.at[iuv], bu, s1)
            cb = pltpu.async_copy(pb_hbm.at[ibv], bb, s2)
            cu.wait()
            cb.wait()

            def rowf(ri, _):
                acc = acc0
                for j in range(8):
                    sl = pl.ds(j * 16, 16)
                    q = jnp.maximum(bu[ri, sl] + bb[ri, sl], 0.0)
                    acc = acc + q * wv[j]
                pbuf[ri] = acc
                return 0

            lax.fori_loop(0, _CH, rowf, 0)
            pltpu.sync_copy(pbuf, out_hbm.at[pl.ds(base, _CH)])
            return 0

        lax.fori_loop(0, nch, ch, 0)

    return k(Pu, Pb, iu, ib, w, bvec)


def _pred_reduce(P, BM=1024):
    L = P.shape[0]

    def body(p_ref, o_ref):
        o_ref[...] = jnp.sum(p_ref[...], axis=1, keepdims=True)

    return pl.pallas_call(
        body,
        grid=(L // BM,),
        in_specs=[pl.BlockSpec((BM, 16), lambda i: (i, 0))],
        out_specs=pl.BlockSpec((BM, 1), lambda i: (i, 0)),
        out_shape=jax.ShapeDtypeStruct((L, 1), jnp.float32),
    )(P)


_ETS = ['ub', 'bu', 'bg', 'gb']
_TOPO = {'ub': ('user', 'book'), 'bu': ('book', 'user'),
         'bg': ('book', 'genre'), 'gb': ('genre', 'book')}


def kernel(params, n_id_user, n_id_book, n_id_genre, src_ub, dst_ub, src_bu,
           dst_bu, src_bg, dst_bg, src_gb, dst_gb, eli_user, eli_book,
           ea_ub, ea_bu, ea_bg, ea_gb):
    p = params
    zeros128 = jnp.zeros((128,), jnp.float32)

    ncnt = {'user': n_id_user.shape[0], 'book': n_id_book.shape[0],
            'genre': n_id_genre.shape[0]}
    npad = {t: ((ncnt[t] + 1 + 255) // 256) * 256 for t in ncnt}

    src_e = {'ub': src_ub, 'bu': src_bu, 'bg': src_bg, 'gb': src_gb}
    dst_e = {'ub': dst_ub, 'bu': dst_bu, 'bg': dst_bg, 'gb': dst_gb}
    ea_e = {'ub': ea_ub, 'bu': ea_bu, 'bg': ea_bg, 'gb': ea_gb}
    ds_pack = {}
    epad = {}
    for e in _ETS:
        E = src_e[e].shape[0]
        Ep = ((E + _NS * _ECH * 2 - 1) // (_NS * _ECH * 2)) * (_NS * _ECH * 2)
        epad[e] = Ep
        d = _TOPO[e][1]
        src_p = _pad_idx(src_e[e], Ep, 0)
        dst_p = _pad_idx(dst_e[e], Ep, ncnt[d])
        ds_pack[e] = jnp.stack([dst_p.reshape(-1, _ECH),
                                src_p.reshape(-1, _ECH)], axis=1)
        ea_e[e] = _pad_rows(ea_e[e], Ep)

    Mu = _mlp2(p['emb_user'], p['init_node']['user']['W1'],
               p['init_node']['user']['b1'], p['init_node']['user']['W2'],
               p['init_node']['user']['b2'], 1000)
    Mb = _mlp2(p['emb_book'], p['init_node']['book']['W1'],
               p['init_node']['book']['b1'], p['init_node']['book']['W2'],
               p['init_node']['book']['b2'], 1000)
    Mg = _mlp2(p['emb_book'], p['init_node']['genre']['W1'],
               p['init_node']['genre']['b1'], p['init_node']['genre']['W2'],
               p['init_node']['genre']['b2'], 1000)
    x = {'user': _gather_rows(Mu, _pad_idx(n_id_user, npad['user'], 0)),
         'book': _gather_rows(Mb, _pad_idx(n_id_book, npad['book'], 0)),
         'genre': _gather_rows(Mg, _pad_idx(n_id_genre, npad['genre'], 0))}

    C = {}
    for e in _ETS:
        ep = p['init_edge'][e]
        Us, vs = [], []
        for li in range(2):
            W1c = p['convs'][li][e]['W1'][256:384]
            Us.append(ep['W2'] @ W1c)
            vs.append(ep['b2'] @ W1c + p['convs'][li][e]['b1'])
        C[e] = _edge_c(ea_e[e], ep['W1'], ep['b1'],
                       jnp.concatenate(Us, axis=1), jnp.concatenate(vs))

    degs = {}
    for li in range(2):
        cw = {e: p['convs'][li][e]['W1'] for e in _ETS}
        b_ub, a_bu = _matmul_multi(
            x['user'], jnp.concatenate([cw['ub'][128:256], cw['bu'][:128]],
                                       axis=1), zeros128.repeat(2), 2, 1024)
        a_ub, b_bu, b_bg, a_gb = _matmul_multi(
            x['book'], jnp.concatenate([cw['ub'][:128], cw['bu'][128:256],
                                        cw['bg'][128:256], cw['gb'][:128]],
                                       axis=1), zeros128.repeat(4), 4, 1024)
        a_bg, b_gb = _matmul_multi(
            x['genre'], jnp.concatenate([cw['bg'][:128], cw['gb'][128:256]],
                                        axis=1), zeros128.repeat(2), 2,
            _chunk(npad['genre'], 1024))
        S = {}
        r1 = _edge_pair((a_ub, b_ub, C['ub'][li], ds_pack['ub'],
                         npad['book']),
                        (a_bu, b_bu, C['bu'][li], ds_pack['bu'],
                         npad['user']), li == 0)
        r2 = _edge_pair((a_gb, b_gb, C['gb'][li], ds_pack['gb'],
                         npad['book']),
                        (a_bg, b_bg, C['bg'][li], ds_pack['bg'],
                         npad['genre']), li == 0)
        S['ub'], S['bu'] = r1[0], r1[1]
        S['gb'], S['bg'] = r2[0], r2[1]
        if li == 0:
            degs['ub'] = r1[2].reshape(-1, 1)
            degs['bu'] = r1[3].reshape(-1, 1)
            degs['gb'] = r2[2].reshape(-1, 1)
            degs['bg'] = r2[3].reshape(-1, 1)
        w2 = {e: p['convs'][li][e]['W2'] for e in _ETS}
        b2 = {e: p['convs'][li][e]['b2'].reshape(1, -1) for e in _ETS}
        relu_out = li < 1
        x = {'book': _update(x['book'], [S['ub'], S['gb']],
                             [w2['ub'], w2['gb']], [degs['ub'], degs['gb']],
                             [b2['ub'], b2['gb']],
                             2.0 * (1.0 + _EPS), relu_out, 1024),
             'user': _update(x['user'], [S['bu']], [w2['bu']], [degs['bu']],
                             [b2['bu']], 1.0 + _EPS, relu_out, 1024),
             'genre': _update(x['genre'], [S['bg']], [w2['bg']], [degs['bg']],
                              [b2['bg']], 1.0 + _EPS, relu_out,
                              _chunk(npad['genre'], 1024))}

    L = eli_user.shape[0]
    Lp = ((L + _NW * _CH - 1) // (_NW * _CH)) * (_NW * _CH)
    Pu, = _matmul_multi(x['user'], p['cls']['W1'][:128], p['cls']['b1'],
                        1, 1024)
    Pb, = _matmul_multi(x['book'], p['cls']['W1'][128:], zeros128, 1, 1024)
    part = _pred_gather(Pu, Pb, _pad_idx(eli_user, Lp, 0),
                        _pad_idx(eli_book, Lp, 0), p['cls']['W2'][:, 0],
                        jnp.full((16,), p['cls']['b2'][0], jnp.float32))
    pred = _pred_reduce(part).reshape(-1)

    return (pred[:L], x['user'][:ncnt['user']], x['book'][:ncnt['book']],
            x['genre'][:ncnt['genre']])
